# baseline (speedup 1.0000x reference)
import functools

import jax
import jax.numpy as jnp
from jax import lax
from jax.experimental import pallas as pl
from jax.experimental.pallas import tpu as pltpu
from jax.experimental.pallas import tpu_sc as plsc

N = 100000
C = 128
G = 64
NW = 32

NSC = 32000
RPW = NSC // NW
B = 125
NBLK = RPW // B
CCH = C // 16
NBUF = 2
IDSB = 1032

NTC = N - NSC
RTC = 2000
TSTEPS = NTC // RTC
TOFF = NSC // RTC


def _sc_partials(feat_flat, ids):
    mesh = plsc.VectorSubcoreMesh(core_axis_name="c", subcore_axis_name="s")

    @functools.partial(
        pl.kernel,
        mesh=mesh,
        out_type=(
            jax.ShapeDtypeStruct((NW * G * C,), jnp.float32),
            jax.ShapeDtypeStruct((NW * G,), jnp.float32),
        ),
        scratch_types=[
            pltpu.VMEM((NBUF, B * C), jnp.float32),
            pltpu.VMEM((IDSB,), jnp.int32),
            pltpu.VMEM((G * C,), jnp.float32),
            pltpu.VMEM((G,), jnp.float32),
        ]
        + [pltpu.SemaphoreType.DMA] * (NBUF + 1),
    )
    def k(feat_hbm, ids_hbm, part_hbm, cnt_hbm,
          bufs, idsb, acc, cnt, *sems):
        semi = sems[NBUF]
        wid = lax.axis_index("s") * 2 + lax.axis_index("c")
        row0 = wid * RPW
        start8 = (row0 // 8) * 8
        off = row0 - start8
        zero = jnp.zeros((16,), jnp.float32)
        iota = lax.iota(jnp.int32, 16)

        ids_cp = pltpu.make_async_copy(
            ids_hbm.at[pl.ds(start8, IDSB)], idsb, semi
        )
        ids_cp.start()

        def feat_cp(blk, q):
            return pltpu.make_async_copy(
                feat_hbm.at[pl.ds((row0 + blk * B) * C, B * C)],
                bufs.at[q], sems[q]
            )

        for q in range(NBUF):
            feat_cp(q, q).start()

        def zero_body(i, _):
            acc[pl.ds(i * 16, 16)] = zero
            return 0

        lax.fori_loop(0, G * C // 16, zero_body, 0)
        for q in range(G // 16):
            cnt[pl.ds(q * 16, 16)] = zero
        ids_cp.wait()

        def compute(blk, q):
            buf = bufs.at[q]
            rbase0 = off + blk * B
            id0 = idsb[pl.ds(rbase0, 16)][0]
            id1 = idsb[pl.ds(rbase0 + B - 16, 16)][15]

            @pl.when(id0 == id1)
            def _uniform():
                def row(r, carry):
                    base = r * C
                    return tuple(
                        carry[c] + buf[pl.ds(base + c * 16, 16)]
                        for c in range(CCH)
                    )

                sums = lax.fori_loop(
                    0, B, row, tuple(zero for _ in range(CCH))
                )
                abase = id0 * C
                for c in range(CCH):
                    sl = pl.ds(abase + c * 16, 16)
                    acc[sl] = acc[sl] + sums[c]
                cbase = (id0 // 16) * 16
                csl = pl.ds(cbase, 16)
                cnt[csl] = cnt[csl] + jnp.where(
                    iota + cbase == id0, float(B), 0.0
                )

            @pl.when(id0 != id1)
            def _boundary():
                def row(r, _):
                    idr = idsb[pl.ds(rbase0 + r, 16)][0]
                    abase = idr * C
                    rbase = r * C
                    for c in range(CCH):
                        sl = pl.ds(abase + c * 16, 16)
                        acc[sl] = acc[sl] + buf[pl.ds(rbase + c * 16, 16)]
                    cbase = (idr // 16) * 16
                    csl = pl.ds(cbase, 16)
                    cnt[csl] = cnt[csl] + jnp.where(
                        iota + cbase == idr, 1.0, 0.0
                    )
                    return 0

                lax.fori_loop(0, B, row, 0)

        def ring_body(p, _):
            for q in range(NBUF):
                blk = p * NBUF + q
                feat_cp(blk, q).wait()
                compute(blk, q)

                @pl.when(blk + NBUF < NBLK)
                def _():
                    feat_cp(blk + NBUF, q).start()

            return 0

        lax.fori_loop(0, NBLK // NBUF, ring_body, 0)
        for q in range(NBLK % NBUF):
            blk = (NBLK // NBUF) * NBUF + q
            feat_cp(blk, q).wait()
            compute(blk, q)

        pltpu.sync_copy(acc, part_hbm.at[pl.ds(wid * G * C, G * C)])
        pltpu.sync_copy(cnt, cnt_hbm.at[pl.ds(wid * G, G)])

    return k(feat_flat, ids)


def _tc_partials(feat2d, ids3d):
    def body(ids_ref, feat_ref, part_ref, cnt_ref):
        i = pl.program_id(0)
        ids_b = ids_ref[0, 0, :]
        gids = lax.broadcasted_iota(jnp.int32, (G, RTC), 0)
        onehot = (ids_b[None, :] == gids).astype(jnp.float32)
        psum = jax.lax.dot(
            onehot, feat_ref[...],
            precision=jax.lax.Precision.HIGHEST,
            preferred_element_type=jnp.float32,
        )
        pcnt = jnp.sum(onehot, axis=1)

        @pl.when(i == 0)
        def _():
            part_ref[...] = psum
            cnt_ref[...] = pcnt

        @pl.when(i > 0)
        def _():
            part_ref[...] = part_ref[...] + psum
            cnt_ref[...] = cnt_ref[...] + pcnt

    return pl.pallas_call(
        body,
        grid=(TSTEPS,),
        in_specs=[
            pl.BlockSpec((1, 1, RTC), lambda i: (TOFF + i, 0, 0)),
            pl.BlockSpec((RTC, C), lambda i: (TOFF + i, 0)),
        ],
        out_specs=[
            pl.BlockSpec((G, C), lambda i: (0, 0)),
            pl.BlockSpec((G,), lambda i: (0,)),
        ],
        out_shape=[
            jax.ShapeDtypeStruct((G, C), jnp.float32),
            jax.ShapeDtypeStruct((G,), jnp.float32),
        ],
    )(ids3d, feat2d)


def _combine(part, cnt, tc_part, tc_cnt):
    def body(part_ref, cnt_ref, tcp_ref, tcc_ref, out_ref):
        sums = jnp.sum(part_ref[...], axis=0) + tcp_ref[...]
        n = jnp.sum(cnt_ref[...], axis=0) + tcc_ref[...]
        n = jnp.maximum(n, 1.0)
        out_ref[...] = sums / n[:, None]

    return pl.pallas_call(
        body,
        out_shape=jax.ShapeDtypeStruct((G, C), jnp.float32),
    )(part, cnt, tc_part, tc_cnt)


def kernel(feat0, graph_ids):
    feat2d = feat0.reshape(N, C)
    feat_flat = feat0.reshape(N * C)
    ids = graph_ids.astype(jnp.int32)
    ids3d = ids.reshape(N // RTC, 1, RTC)
    part, cnt = _sc_partials(feat_flat, ids)
    tc_part, tc_cnt = _tc_partials(feat2d, ids3d)
    return _combine(
        part.reshape(NW, G, C), cnt.reshape(NW, G), tc_part, tc_cnt
    )


# speedup vs baseline: 1.2483x; 1.2483x over previous
import functools

import jax
import jax.numpy as jnp
from jax import lax
from jax.experimental import pallas as pl
from jax.experimental.pallas import tpu as pltpu
from jax.experimental.pallas import tpu_sc as plsc

N = 100000
C = 128
G = 64
NW = 32

NSC = 48000
RPW = NSC // NW
B = 125
NBLK = RPW // B
CCH = C // 16
NBUF = 2
IDSB = 1528

NTC = N - NSC
RTC = 2000
TSTEPS = NTC // RTC
TOFF = NSC // RTC


def _sc_partials(feat_flat, ids):
    mesh = plsc.VectorSubcoreMesh(core_axis_name="c", subcore_axis_name="s")

    @functools.partial(
        pl.kernel,
        mesh=mesh,
        out_type=(
            jax.ShapeDtypeStruct((NW * G * C,), jnp.float32),
            jax.ShapeDtypeStruct((NW * G,), jnp.float32),
        ),
        scratch_types=[
            pltpu.VMEM((NBUF, B * C), jnp.float32),
            pltpu.VMEM((IDSB,), jnp.int32),
            pltpu.VMEM((G * C,), jnp.float32),
            pltpu.VMEM((G,), jnp.float32),
        ]
        + [pltpu.SemaphoreType.DMA] * (NBUF + 1),
    )
    def k(feat_hbm, ids_hbm, part_hbm, cnt_hbm,
          bufs, idsb, acc, cnt, *sems):
        semi = sems[NBUF]
        wid = lax.axis_index("s") * 2 + lax.axis_index("c")
        row0 = wid * RPW
        start8 = (row0 // 8) * 8
        off = row0 - start8
        zero = jnp.zeros((16,), jnp.float32)
        iota = lax.iota(jnp.int32, 16)

        ids_cp = pltpu.make_async_copy(
            ids_hbm.at[pl.ds(start8, IDSB)], idsb, semi
        )
        ids_cp.start()

        def feat_cp(blk, q):
            return pltpu.make_async_copy(
                feat_hbm.at[pl.ds((row0 + blk * B) * C, B * C)],
                bufs.at[q], sems[q]
            )

        for q in range(NBUF):
            feat_cp(q, q).start()

        def zero_body(i, _):
            acc[pl.ds(i * 16, 16)] = zero
            return 0

        lax.fori_loop(0, G * C // 16, zero_body, 0)
        for q in range(G // 16):
            cnt[pl.ds(q * 16, 16)] = zero
        ids_cp.wait()

        def compute(blk, q):
            buf = bufs.at[q]
            rbase0 = off + blk * B
            id0 = idsb[pl.ds(rbase0, 16)][0]
            id1 = idsb[pl.ds(rbase0 + B - 16, 16)][15]

            @pl.when(id0 == id1)
            def _uniform():
                def row(r, carry):
                    base = r * C
                    return tuple(
                        carry[c] + buf[pl.ds(base + c * 16, 16)]
                        for c in range(CCH)
                    )

                sums = lax.fori_loop(
                    0, B, row, tuple(zero for _ in range(CCH))
                )
                abase = id0 * C
                for c in range(CCH):
                    sl = pl.ds(abase + c * 16, 16)
                    acc[sl] = acc[sl] + sums[c]
                cbase = (id0 // 16) * 16
                csl = pl.ds(cbase, 16)
                cnt[csl] = cnt[csl] + jnp.where(
                    iota + cbase == id0, float(B), 0.0
                )

            @pl.when(id0 != id1)
            def _boundary():
                def row(r, _):
                    idr = idsb[pl.ds(rbase0 + r, 16)][0]
                    abase = idr * C
                    rbase = r * C
                    for c in range(CCH):
                        sl = pl.ds(abase + c * 16, 16)
                        acc[sl] = acc[sl] + buf[pl.ds(rbase + c * 16, 16)]
                    cbase = (idr // 16) * 16
                    csl = pl.ds(cbase, 16)
                    cnt[csl] = cnt[csl] + jnp.where(
                        iota + cbase == idr, 1.0, 0.0
                    )
                    return 0

                lax.fori_loop(0, B, row, 0)

        def ring_body(p, _):
            for q in range(NBUF):
                blk = p * NBUF + q
                feat_cp(blk, q).wait()
                compute(blk, q)

                @pl.when(blk + NBUF < NBLK)
                def _():
                    feat_cp(blk + NBUF, q).start()

            return 0

        lax.fori_loop(0, NBLK // NBUF, ring_body, 0)
        for q in range(NBLK % NBUF):
            blk = (NBLK // NBUF) * NBUF + q
            feat_cp(blk, q).wait()
            compute(blk, q)

        pltpu.sync_copy(acc, part_hbm.at[pl.ds(wid * G * C, G * C)])
        pltpu.sync_copy(cnt, cnt_hbm.at[pl.ds(wid * G, G)])

    return k(feat_flat, ids)


def _tc_partials(feat2d, ids3d):
    def body(ids_ref, feat_ref, part_ref, cnt_ref):
        i = pl.program_id(0)
        ids_b = ids_ref[0, 0, :]
        gids = lax.broadcasted_iota(jnp.int32, (G, RTC), 0)
        onehot = (ids_b[None, :] == gids).astype(jnp.float32)
        psum = jax.lax.dot(
            onehot, feat_ref[...],
            preferred_element_type=jnp.float32,
        )
        pcnt = jnp.sum(onehot, axis=1)

        @pl.when(i == 0)
        def _():
            part_ref[...] = psum
            cnt_ref[...] = pcnt

        @pl.when(i > 0)
        def _():
            part_ref[...] = part_ref[...] + psum
            cnt_ref[...] = cnt_ref[...] + pcnt

    return pl.pallas_call(
        body,
        grid=(TSTEPS,),
        in_specs=[
            pl.BlockSpec((1, 1, RTC), lambda i: (TOFF + i, 0, 0)),
            pl.BlockSpec((RTC, C), lambda i: (TOFF + i, 0)),
        ],
        out_specs=[
            pl.BlockSpec((G, C), lambda i: (0, 0)),
            pl.BlockSpec((G,), lambda i: (0,)),
        ],
        out_shape=[
            jax.ShapeDtypeStruct((G, C), jnp.float32),
            jax.ShapeDtypeStruct((G,), jnp.float32),
        ],
    )(ids3d, feat2d)


def _combine(part, cnt, tc_part, tc_cnt):
    def body(part_ref, cnt_ref, tcp_ref, tcc_ref, out_ref):
        sums = jnp.sum(part_ref[...], axis=0) + tcp_ref[...]
        n = jnp.sum(cnt_ref[...], axis=0) + tcc_ref[...]
        n = jnp.maximum(n, 1.0)
        out_ref[...] = sums / n[:, None]

    return pl.pallas_call(
        body,
        out_shape=jax.ShapeDtypeStruct((G, C), jnp.float32),
    )(part, cnt, tc_part, tc_cnt)


def kernel(feat0, graph_ids):
    feat2d = feat0.reshape(N, C)
    feat_flat = feat0.reshape(N * C)
    ids = graph_ids.astype(jnp.int32)
    ids3d = ids.reshape(N // RTC, 1, RTC)
    part, cnt = _sc_partials(feat_flat, ids)
    tc_part, tc_cnt = _tc_partials(feat2d, ids3d)
    return _combine(
        part.reshape(NW, G, C), cnt.reshape(NW, G), tc_part, tc_cnt
    )
